# Initial kernel scaffold; baseline (speedup 1.0000x reference)
import jax
import jax.numpy as jnp
from jax.experimental import pallas as pl


def kernel(x, edge_index, W1, a_src1, a_dst1, W2, a_src2, a_dst2):
    raise NotImplementedError("write your pallas kernel here")


# speedup vs baseline: 1.0810x; 1.0810x over previous
import jax
import jax.numpy as jnp
from jax.experimental import pallas as pl

N_NODES = 10000
HEADS = 8
HIDDEN = 8
N_CLASSES = 16


def _gat(x, W, a_src, a_dst, src, dst, n, H):
    h = (x @ W).reshape(n, H, -1)
    asrc = jnp.sum(h * a_src[None], -1)
    adst = jnp.sum(h * a_dst[None], -1)
    t = jnp.max(asrc, axis=0) + adst
    mub = jnp.where(t > 0, t, 0.2 * t)
    z = asrc[src] + adst[dst]
    e = jnp.where(z > 0, z, 0.2 * z)
    ex = jnp.exp(e - mub[dst])
    num = jax.ops.segment_sum(ex[:, :, None] * h[src], dst, num_segments=n)
    den = jax.ops.segment_sum(ex, dst, num_segments=n)
    return num, den


def _div_kernel(num_ref, den_ref, out_ref):
    out_ref[...] = num_ref[...] / (den_ref[...] + 1e-9)


def kernel(x, edge_index, W1, a_src1, a_dst1, W2, a_src2, a_dst2):
    src, dst = edge_index[0], edge_index[1]
    num1, den1 = _gat(x, W1, a_src1, a_dst1, src, dst, N_NODES, HEADS)
    h1 = num1 / (den1[:, :, None] + 1e-9)
    h1 = jax.nn.elu(h1).reshape(N_NODES, HEADS * HIDDEN)
    num2, den2 = _gat(h1, W2, a_src2, a_dst2, src, dst, N_NODES, 1)
    num2 = num2.reshape(N_NODES, N_CLASSES)
    den2 = jnp.broadcast_to(den2, (N_NODES, N_CLASSES))
    out = pl.pallas_call(
        _div_kernel,
        out_shape=jax.ShapeDtypeStruct((N_NODES, N_CLASSES), jnp.float32),
    )(num2, den2)
    return out


# speedup vs baseline: 44.8569x; 41.4951x over previous
import functools

import jax
import jax.numpy as jnp
from jax import lax
from jax.experimental import pallas as pl
from jax.experimental.pallas import tpu as pltpu
from jax.experimental.pallas import tpu_sc as plsc

N = 10000
E = 320000
D = 128
H1 = 8
F1 = 8
C2 = 16

NC = 2
NS = 16
NW = NC * NS
EW = E // NW
CH = 80
NCH = EW // CH
NP = 10240
SEG = NP // NS
SR1 = 80
DR = 16
SR2 = 32

_mesh = plsc.VectorSubcoreMesh(
    core_axis_name="c", subcore_axis_name="s", num_cores=NC, num_subcores=NS)


def _tabs1_body(x_ref, w_ref, asb_ref, adb_ref, st_ref, dt_ref):
    h = jnp.dot(x_ref[...], w_ref[...], preferred_element_type=jnp.float32)
    asrc = jnp.dot(h, asb_ref[...], preferred_element_type=jnp.float32)
    adst = jnp.dot(h, adb_ref[...], preferred_element_type=jnp.float32)
    t = jnp.max(asrc, axis=0, keepdims=True) + adst
    mub = jnp.where(t > 0, t, 0.2 * t)
    st_ref[...] = jnp.concatenate(
        [h, asrc, jnp.zeros((N, SR1 - H1 * F1 - H1), jnp.float32)], axis=1)
    dt_ref[...] = jnp.concatenate([adst, mub], axis=1)


def _tabs2_body(acc_ref, w2_ref, as2_ref, ad2_ref, rep_ref, st_ref, dt_ref):
    acc = acc_ref[0][:N] + acc_ref[1][:N]
    num = acc[:, 0:64]
    den = jnp.dot(acc[:, 64:72], rep_ref[...], preferred_element_type=jnp.float32)
    h1 = num / (den + 1e-9)
    h1 = jnp.where(h1 > 0, h1, jnp.exp(jnp.minimum(h1, 0.0)) - 1.0)
    h2 = jnp.dot(h1, w2_ref[...], preferred_element_type=jnp.float32)
    asrc2 = jnp.sum(h2 * as2_ref[...], axis=1, keepdims=True)
    adst2 = jnp.sum(h2 * ad2_ref[...], axis=1, keepdims=True)
    t = jnp.max(asrc2, axis=0, keepdims=True) + adst2
    mub2 = jnp.where(t > 0, t, 0.2 * t)
    st_ref[...] = jnp.concatenate(
        [h2, asrc2, jnp.zeros((N, SR2 - C2 - 1), jnp.float32)], axis=1)
    dt_ref[...] = jnp.concatenate(
        [adst2, mub2, jnp.zeros((N, DR - 2), jnp.float32)], axis=1)


def _final_body(acc_ref, out_ref):
    acc = acc_ref[0][:N] + acc_ref[1][:N]
    out_ref[...] = acc[:, 0:C2] / (acc[:, C2:C2 + 1] + 1e-9)


def _zero_acc(bounce, acc, sid, width):
    zero16 = jnp.zeros((16,), jnp.float32)

    def zrow(i, carry):
        for k in range(width // 16):
            bounce[i, pl.ds(16 * k, 16)] = zero16
        return carry

    lax.fori_loop(0, 128, zrow, 0)
    for b in range(5):
        pltpu.sync_copy(bounce, acc.at[pl.ds(sid * SEG + b * 128, 128)])


def _dump_acc(bounce, acc, out_hbm, cid, sid):
    for b in range(5):
        pltpu.sync_copy(acc.at[pl.ds(sid * SEG + b * 128, 128)], bounce)
        pltpu.sync_copy(bounce, out_hbm.at[cid, pl.ds(sid * SEG + b * 128, 128)])


def _edge1_body(src_hbm, dst_hbm, st_hbm, dt_hbm, out_hbm,
                ids_s, ids_d, srows, drows, msg, exbuf, bounce, acc,
                sem1, sem2):
    cid = lax.axis_index("c")
    sid = lax.axis_index("s")
    _zero_acc(bounce, acc, sid, SR1)
    plsc.subcore_barrier()

    iota = lax.iota(jnp.int32, 16)
    lo8 = iota < 8
    colm8 = jnp.where(lo8, iota, iota - 8)
    col_asrc = 64 + colm8
    col_adst = colm8
    col_mub = 8 + colm8

    def chunk(j, carry):
        base = cid * (NS * EW) + sid * EW + j * CH
        pltpu.sync_copy(src_hbm.at[pl.ds(base, CH)], ids_s)
        pltpu.sync_copy(dst_hbm.at[pl.ds(base, CH)], ids_d)
        cp1 = pltpu.async_copy(st_hbm.at[ids_s], srows, sem1)
        cp2 = pltpu.async_copy(dt_hbm.at[ids_d], drows, sem2)
        cp1.wait()
        cp2.wait()

        def pair(p, c2):
            ea = 2 * p
            eb = ea + 1
            rows = jnp.where(lo8, ea, eb)
            sab = plsc.load_gather(srows, [rows, col_asrc])
            dab = plsc.load_gather(drows, [rows, col_adst])
            mab = plsc.load_gather(drows, [rows, col_mub])
            z = sab + dab
            ex = jnp.exp(jnp.maximum(z, 0.2 * z) - mab)
            exbuf[...] = ex
            for lanebase, ev in ((0, ea), (8, eb)):
                for k in range(4):
                    gi = jnp.where(lo8, lanebase + 2 * k, lanebase + 2 * k + 1)
                    exk = plsc.load_gather(exbuf, [gi])
                    msg[ev, pl.ds(16 * k, 16)] = exk * srows[ev, pl.ds(16 * k, 16)]
                gd = jnp.where(lo8, lanebase + iota, 0)
                exd = plsc.load_gather(exbuf, [gd])
                msg[ev, pl.ds(64, 16)] = jnp.where(lo8, exd, 0.0)
            return c2

        lax.fori_loop(0, CH // 2, pair, 0)
        pltpu.sync_copy(msg, acc.at[ids_d], add=True)
        return carry

    lax.fori_loop(0, NCH, chunk, 0)
    plsc.subcore_barrier()
    _dump_acc(bounce, acc, out_hbm, cid, sid)


def _edge2_body(src_hbm, dst_hbm, st_hbm, dt_hbm, out_hbm,
                ids_s, ids_d, srows, drows, msg, exbuf, bounce, acc,
                sem1, sem2):
    cid = lax.axis_index("c")
    sid = lax.axis_index("s")
    _zero_acc(bounce, acc, sid, SR2)
    plsc.subcore_barrier()

    iota = lax.iota(jnp.int32, 16)
    c_as = jnp.full((16,), C2, jnp.int32)
    c_ad = jnp.zeros((16,), jnp.int32)
    c_mu = jnp.ones((16,), jnp.int32)

    def zpad(i, carry):
        msg[i, pl.ds(16, 16)] = jnp.zeros((16,), jnp.float32)
        return carry

    lax.fori_loop(0, CH, zpad, 0)

    def chunk(j, carry):
        base = cid * (NS * EW) + sid * EW + j * CH
        pltpu.sync_copy(src_hbm.at[pl.ds(base, CH)], ids_s)
        pltpu.sync_copy(dst_hbm.at[pl.ds(base, CH)], ids_d)
        cp1 = pltpu.async_copy(st_hbm.at[ids_s], srows, sem1)
        cp2 = pltpu.async_copy(dt_hbm.at[ids_d], drows, sem2)
        cp1.wait()
        cp2.wait()

        def grp(g, c2):
            rows = 16 * g + iota
            s16 = plsc.load_gather(srows, [rows, c_as])
            d16 = plsc.load_gather(drows, [rows, c_ad])
            m16 = plsc.load_gather(drows, [rows, c_mu])
            z = s16 + d16
            ex = jnp.exp(jnp.maximum(z, 0.2 * z) - m16)
            plsc.store_scatter(msg, [rows, c_as], ex)
            for f in range(C2):
                cf = jnp.full((16,), f, jnp.int32)
                col = plsc.load_gather(srows, [rows, cf])
                plsc.store_scatter(msg, [rows, cf], ex * col)
            return c2

        lax.fori_loop(0, CH // 16, grp, 0)
        pltpu.sync_copy(msg, acc.at[ids_d], add=True)
        return carry

    lax.fori_loop(0, NCH, chunk, 0)
    plsc.subcore_barrier()
    _dump_acc(bounce, acc, out_hbm, cid, sid)


_edge1 = functools.partial(
    pl.kernel, _edge1_body,
    out_type=jax.ShapeDtypeStruct((NC, NP, SR1), jnp.float32),
    mesh=_mesh,
    compiler_params=pltpu.CompilerParams(needs_layout_passes=False, use_tc_tiling_on_sc=False),
    scratch_types=[
        pltpu.VMEM((CH,), jnp.int32),
        pltpu.VMEM((CH,), jnp.int32),
        pltpu.VMEM((CH, SR1), jnp.float32),
        pltpu.VMEM((CH, DR), jnp.float32),
        pltpu.VMEM((CH, SR1), jnp.float32),
        pltpu.VMEM((16,), jnp.float32),
        pltpu.VMEM((128, SR1), jnp.float32),
        pltpu.VMEM_SHARED((NP, SR1), jnp.float32),
        pltpu.SemaphoreType.DMA,
        pltpu.SemaphoreType.DMA,
    ],
)()

_edge2 = functools.partial(
    pl.kernel, _edge2_body,
    out_type=jax.ShapeDtypeStruct((NC, NP, SR2), jnp.float32),
    mesh=_mesh,
    compiler_params=pltpu.CompilerParams(needs_layout_passes=False, use_tc_tiling_on_sc=False),
    scratch_types=[
        pltpu.VMEM((CH,), jnp.int32),
        pltpu.VMEM((CH,), jnp.int32),
        pltpu.VMEM((CH, SR2), jnp.float32),
        pltpu.VMEM((CH, DR), jnp.float32),
        pltpu.VMEM((CH, SR2), jnp.float32),
        pltpu.VMEM((16,), jnp.float32),
        pltpu.VMEM((128, SR2), jnp.float32),
        pltpu.VMEM_SHARED((NP, SR2), jnp.float32),
        pltpu.SemaphoreType.DMA,
        pltpu.SemaphoreType.DMA,
    ],
)()


def kernel(x, edge_index, W1, a_src1, a_dst1, W2, a_src2, a_dst2):
    src = edge_index[0]
    dst = edge_index[1]

    ii = jnp.arange(H1 * F1)
    asb = jnp.zeros((H1 * F1, H1), jnp.float32).at[ii, ii // F1].set(a_src1.reshape(-1))
    adb = jnp.zeros((H1 * F1, H1), jnp.float32).at[ii, ii // F1].set(a_dst1.reshape(-1))
    rep = jnp.zeros((H1, H1 * F1), jnp.float32).at[ii // F1, ii].set(1.0)

    st1, dt1 = pl.pallas_call(
        _tabs1_body,
        out_shape=(
            jax.ShapeDtypeStruct((N, SR1), jnp.float32),
            jax.ShapeDtypeStruct((N, DR), jnp.float32),
        ),
    )(x, W1, asb, adb)

    acc1 = _edge1(src, dst, st1, dt1)

    st2, dt2 = pl.pallas_call(
        _tabs2_body,
        out_shape=(
            jax.ShapeDtypeStruct((N, SR2), jnp.float32),
            jax.ShapeDtypeStruct((N, DR), jnp.float32),
        ),
    )(acc1, W2, a_src2, a_dst2, rep)

    acc2 = _edge2(src, dst, st2, dt2)

    out = pl.pallas_call(
        _final_body,
        out_shape=jax.ShapeDtypeStruct((N, C2), jnp.float32),
    )(acc2)
    return out


# speedup vs baseline: 61.9187x; 1.3804x over previous
import functools

import jax
import jax.numpy as jnp
from jax import lax
from jax.experimental import pallas as pl
from jax.experimental.pallas import tpu as pltpu
from jax.experimental.pallas import tpu_sc as plsc

N = 10000
E = 320000
D = 128
H1 = 8
F1 = 8
C2 = 16

NC = 2
NS = 16
NW = NC * NS
EW = E // NW
CH = 80
NCH = EW // CH
NP = 10240
SEG = NP // NS
SR1 = 80
DR = 16
SR2 = 32

_mesh = plsc.VectorSubcoreMesh(
    core_axis_name="c", subcore_axis_name="s", num_cores=NC, num_subcores=NS)
_params = pltpu.CompilerParams(
    needs_layout_passes=False, use_tc_tiling_on_sc=False)


def _tabs1_body(x_ref, w_ref, asb_ref, adb_ref, st_ref, dt_ref):
    h = jnp.dot(x_ref[...], w_ref[...], preferred_element_type=jnp.float32)
    asrc = jnp.dot(h, asb_ref[...], preferred_element_type=jnp.float32)
    adst = jnp.dot(h, adb_ref[...], preferred_element_type=jnp.float32)
    t = jnp.max(asrc, axis=0, keepdims=True) + adst
    mub = jnp.where(t > 0, t, 0.2 * t)
    st_ref[...] = jnp.concatenate(
        [h, asrc, jnp.zeros((N, SR1 - H1 * F1 - H1), jnp.float32)], axis=1)
    dt_ref[...] = jnp.concatenate([adst, mub], axis=1)


def _tabs2_body(acc_ref, w2_ref, as2_ref, ad2_ref, rep_ref, st_ref, dt_ref):
    acc = acc_ref[0][:N] + acc_ref[1][:N]
    num = acc[:, 0:64]
    den = jnp.dot(acc[:, 64:72], rep_ref[...], preferred_element_type=jnp.float32)
    h1 = num / (den + 1e-9)
    h1 = jnp.where(h1 > 0, h1, jnp.exp(jnp.minimum(h1, 0.0)) - 1.0)
    h2 = jnp.dot(h1, w2_ref[...], preferred_element_type=jnp.float32)
    asrc2 = jnp.sum(h2 * as2_ref[...], axis=1, keepdims=True)
    adst2 = jnp.sum(h2 * ad2_ref[...], axis=1, keepdims=True)
    t = jnp.max(asrc2, axis=0, keepdims=True) + adst2
    mub2 = jnp.where(t > 0, t, 0.2 * t)
    st_ref[...] = jnp.concatenate(
        [h2, asrc2, jnp.zeros((N, SR2 - C2 - 1), jnp.float32)], axis=1)
    dt_ref[...] = jnp.concatenate(
        [adst2, mub2, jnp.zeros((N, DR - 2), jnp.float32)], axis=1)


def _final_body(acc_ref, out_ref):
    acc = acc_ref[0][:N] + acc_ref[1][:N]
    out_ref[...] = acc[:, 0:C2] / (acc[:, C2:C2 + 1] + 1e-9)


def _zero_acc(bounce, acc, sid, width):
    zero16 = jnp.zeros((16,), jnp.float32)

    def zrow(i, carry):
        for k in range(width // 16):
            bounce[i, pl.ds(16 * k, 16)] = zero16
        return carry

    lax.fori_loop(0, 128, zrow, 0)
    for b in range(5):
        pltpu.sync_copy(bounce, acc.at[pl.ds(sid * SEG + b * 128, 128)])


def _dump_acc(bounce, acc, out_hbm, cid, sid):
    for b in range(5):
        pltpu.sync_copy(acc.at[pl.ds(sid * SEG + b * 128, 128)], bounce)
        pltpu.sync_copy(bounce, out_hbm.at[cid, pl.ds(sid * SEG + b * 128, 128)])


def _edge_pipeline(src3, dst3, st_hbm, dt_hbm, out_hbm, ids_s, ids_d,
                   srows2, drows2, msg2, bounce, acc, gs, gd, sc,
                   width, compute):
    cid = lax.axis_index("c")
    sid = lax.axis_index("s")
    wid = cid * NS + sid
    _zero_acc(bounce, acc, sid, width)

    zero16 = jnp.zeros((16,), jnp.float32)

    def zpad2(i, carry):
        for b in range(2):
            msg2[b][i, pl.ds(width - 16, 16)] = zero16
        return carry

    lax.fori_loop(0, CH, zpad2, 0)

    pltpu.sync_copy(src3.at[wid], ids_s)
    pltpu.sync_copy(dst3.at[wid], ids_d)
    plsc.subcore_barrier()

    def gdesc(b, j):
        return (pltpu.make_async_copy(st_hbm.at[ids_s.at[j]], srows2[b], gs[b]),
                pltpu.make_async_copy(dt_hbm.at[ids_d.at[j]], drows2[b], gd[b]))

    def sdesc(b, j):
        return pltpu.make_async_copy(msg2[b], acc.at[ids_d.at[j]], sc[b])

    for b in range(2):
        d1, d2 = gdesc(b, b)
        d1.start()
        d2.start()

    def step(jj, carry):
        for b in range(2):
            j = 2 * jj + b
            d1, d2 = gdesc(b, j)
            d1.wait()
            d2.wait()

            @pl.when(jj > 0)
            def _():
                sdesc(b, j - 2).wait()

            compute(srows2[b], drows2[b], msg2[b])
            sdesc(b, j).start(add=True)

            @pl.when(j + 2 < NCH)
            def _():
                e1, e2 = gdesc(b, j + 2)
                e1.start()
                e2.start()
        return carry

    lax.fori_loop(0, (NCH - 1) // 2, step, 0)
    d1, d2 = gdesc(0, NCH - 1)
    d1.wait()
    d2.wait()
    sdesc(0, NCH - 3).wait()
    compute(srows2[0], drows2[0], msg2[0])
    sdesc(0, NCH - 1).start(add=True)
    sdesc(1, NCH - 2).wait()
    sdesc(0, NCH - 1).wait()

    plsc.subcore_barrier()
    _dump_acc(bounce, acc, out_hbm, cid, sid)


def _compute1(srows, drows, msg):
    iota = lax.iota(jnp.int32, 16)

    def grp(g, carry):
        rows = 16 * g + iota
        for h in range(H1):
            s_h = plsc.load_gather(srows, [rows, jnp.full((16,), 64 + h, jnp.int32)])
            d_h = plsc.load_gather(drows, [rows, jnp.full((16,), h, jnp.int32)])
            m_h = plsc.load_gather(drows, [rows, jnp.full((16,), 8 + h, jnp.int32)])
            z = s_h + d_h
            exh = jnp.exp(jnp.maximum(z, 0.2 * z) - m_h)
            plsc.store_scatter(msg, [rows, jnp.full((16,), 64 + h, jnp.int32)], exh)
            for f in range(F1):
                cf = jnp.full((16,), F1 * h + f, jnp.int32)
                col = plsc.load_gather(srows, [rows, cf])
                plsc.store_scatter(msg, [rows, cf], exh * col)
        return carry

    lax.fori_loop(0, CH // 16, grp, 0)


def _compute2(srows, drows, msg):
    iota = lax.iota(jnp.int32, 16)
    c_as = jnp.full((16,), C2, jnp.int32)
    c_ad = jnp.zeros((16,), jnp.int32)
    c_mu = jnp.ones((16,), jnp.int32)

    def grp(g, carry):
        rows = 16 * g + iota
        s16 = plsc.load_gather(srows, [rows, c_as])
        d16 = plsc.load_gather(drows, [rows, c_ad])
        m16 = plsc.load_gather(drows, [rows, c_mu])
        z = s16 + d16
        ex = jnp.exp(jnp.maximum(z, 0.2 * z) - m16)
        plsc.store_scatter(msg, [rows, c_as], ex)
        for f in range(C2):
            cf = jnp.full((16,), f, jnp.int32)
            col = plsc.load_gather(srows, [rows, cf])
            plsc.store_scatter(msg, [rows, cf], ex * col)
        return carry

    lax.fori_loop(0, CH // 16, grp, 0)


def _edge1_body(src3, dst3, st_hbm, dt_hbm, out_hbm,
                ids_s, ids_d, sr_a, sr_b, dr_a, dr_b, ms_a, ms_b,
                bounce, acc, gs_a, gs_b, gd_a, gd_b, sc_a, sc_b):
    _edge_pipeline(src3, dst3, st_hbm, dt_hbm, out_hbm, ids_s, ids_d,
                   [sr_a, sr_b], [dr_a, dr_b], [ms_a, ms_b], bounce, acc,
                   [gs_a, gs_b], [gd_a, gd_b], [sc_a, sc_b], SR1, _compute1)


def _edge2_body(src3, dst3, st_hbm, dt_hbm, out_hbm,
                ids_s, ids_d, sr_a, sr_b, dr_a, dr_b, ms_a, ms_b,
                bounce, acc, gs_a, gs_b, gd_a, gd_b, sc_a, sc_b):
    _edge_pipeline(src3, dst3, st_hbm, dt_hbm, out_hbm, ids_s, ids_d,
                   [sr_a, sr_b], [dr_a, dr_b], [ms_a, ms_b], bounce, acc,
                   [gs_a, gs_b], [gd_a, gd_b], [sc_a, sc_b], SR2, _compute2)


def _edge_scratch(width):
    return [
        pltpu.VMEM((NCH, CH), jnp.int32),
        pltpu.VMEM((NCH, CH), jnp.int32),
        pltpu.VMEM((CH, width), jnp.float32),
        pltpu.VMEM((CH, width), jnp.float32),
        pltpu.VMEM((CH, DR), jnp.float32),
        pltpu.VMEM((CH, DR), jnp.float32),
        pltpu.VMEM((CH, width), jnp.float32),
        pltpu.VMEM((CH, width), jnp.float32),
        pltpu.VMEM((128, width), jnp.float32),
        pltpu.VMEM_SHARED((NP, width), jnp.float32),
        pltpu.SemaphoreType.DMA,
        pltpu.SemaphoreType.DMA,
        pltpu.SemaphoreType.DMA,
        pltpu.SemaphoreType.DMA,
        pltpu.SemaphoreType.DMA,
        pltpu.SemaphoreType.DMA,
    ]


_edge1 = functools.partial(
    pl.kernel, _edge1_body,
    out_type=jax.ShapeDtypeStruct((NC, NP, SR1), jnp.float32),
    mesh=_mesh,
    compiler_params=_params,
    scratch_types=_edge_scratch(SR1),
)()

_edge2 = functools.partial(
    pl.kernel, _edge2_body,
    out_type=jax.ShapeDtypeStruct((NC, NP, SR2), jnp.float32),
    mesh=_mesh,
    compiler_params=_params,
    scratch_types=_edge_scratch(SR2),
)()


def kernel(x, edge_index, W1, a_src1, a_dst1, W2, a_src2, a_dst2):
    src3 = edge_index[0].reshape(NW, NCH, CH)
    dst3 = edge_index[1].reshape(NW, NCH, CH)

    ii = jnp.arange(H1 * F1)
    asb = jnp.zeros((H1 * F1, H1), jnp.float32).at[ii, ii // F1].set(a_src1.reshape(-1))
    adb = jnp.zeros((H1 * F1, H1), jnp.float32).at[ii, ii // F1].set(a_dst1.reshape(-1))
    rep = jnp.zeros((H1, H1 * F1), jnp.float32).at[ii // F1, ii].set(1.0)

    st1, dt1 = pl.pallas_call(
        _tabs1_body,
        out_shape=(
            jax.ShapeDtypeStruct((N, SR1), jnp.float32),
            jax.ShapeDtypeStruct((N, DR), jnp.float32),
        ),
    )(x, W1, asb, adb)

    acc1 = _edge1(src3, dst3, st1, dt1)

    st2, dt2 = pl.pallas_call(
        _tabs2_body,
        out_shape=(
            jax.ShapeDtypeStruct((N, SR2), jnp.float32),
            jax.ShapeDtypeStruct((N, DR), jnp.float32),
        ),
    )(acc1, W2, a_src2, a_dst2, rep)

    acc2 = _edge2(src3, dst3, st2, dt2)

    out = pl.pallas_call(
        _final_body,
        out_shape=jax.ShapeDtypeStruct((N, C2), jnp.float32),
    )(acc2)
    return out


# speedup vs baseline: 65.8615x; 1.0637x over previous
import functools

import jax
import jax.numpy as jnp
from jax import lax
from jax.experimental import pallas as pl
from jax.experimental.pallas import tpu as pltpu
from jax.experimental.pallas import tpu_sc as plsc

N = 10000
E = 320000
D = 128
H1 = 8
F1 = 8
C2 = 16

NC = 2
NS = 16
NW = NC * NS
EW = E // NW
CH = 80
NCH = EW // CH
NP = 10240
SEG = NP // NS
SR1 = 80
SR2 = 16
AR2 = 32

_mesh = plsc.VectorSubcoreMesh(
    core_axis_name="c", subcore_axis_name="s", num_cores=NC, num_subcores=NS)
_params = pltpu.CompilerParams(
    needs_layout_passes=False, use_tc_tiling_on_sc=False)


def _tabs1_body(x_ref, w_ref, asb_ref, adb_ref, st_ref, dt_ref):
    h = jnp.dot(x_ref[...], w_ref[...], preferred_element_type=jnp.float32)
    asrc = jnp.dot(h, asb_ref[...], preferred_element_type=jnp.float32)
    adst = jnp.dot(h, adb_ref[...], preferred_element_type=jnp.float32)
    t = jnp.max(asrc, axis=0, keepdims=True) + adst
    mub = jnp.where(t > 0, t, 0.2 * t)
    st_ref[...] = jnp.concatenate(
        [h, asrc, jnp.zeros((N, SR1 - H1 * F1 - H1), jnp.float32)], axis=1)
    dt_ref[...] = jnp.concatenate([adst, mub], axis=1)


def _tabs2_body(acc_ref, w2_ref, as2_ref, ad2_ref, rep_ref, st_ref, v3_ref):
    acc = acc_ref[0][:N] + acc_ref[1][:N]
    num = acc[:, 0:64]
    den = jnp.dot(acc[:, 64:72], rep_ref[...], preferred_element_type=jnp.float32)
    h1 = num / (den + 1e-9)
    h1 = jnp.where(h1 > 0, h1, jnp.exp(jnp.minimum(h1, 0.0)) - 1.0)
    h2 = jnp.dot(h1, w2_ref[...], preferred_element_type=jnp.float32)
    asrc2 = lax.dot_general(as2_ref[...], h2, (((1,), (1,)), ((), ())),
                            preferred_element_type=jnp.float32)
    adst2 = lax.dot_general(ad2_ref[...], h2, (((1,), (1,)), ((), ())),
                            preferred_element_type=jnp.float32)
    t = jnp.max(asrc2, axis=1, keepdims=True) + adst2
    mub2 = jnp.where(t > 0, t, 0.2 * t)
    st_ref[...] = h2
    v3_ref[...] = jnp.concatenate([asrc2, adst2, mub2], axis=0)


def _final_body(acc_ref, out_ref):
    acc = acc_ref[0][:N] + acc_ref[1][:N]
    out_ref[...] = acc[:, 0:C2] / (acc[:, C2:C2 + 1] + 1e-9)


def _zero_acc(bounce, acc, sid):
    zero16 = jnp.zeros((16,), jnp.float32)
    width = bounce.shape[1]

    def zrow(i, carry):
        for k in range(width // 16):
            bounce[i, pl.ds(16 * k, 16)] = zero16
        return carry

    lax.fori_loop(0, 128, zrow, 0)
    for b in range(5):
        pltpu.sync_copy(bounce, acc.at[pl.ds(sid * SEG + b * 128, 128)])


def _dump_acc(bounce, acc, out_hbm, cid, sid):
    for b in range(5):
        pltpu.sync_copy(acc.at[pl.ds(sid * SEG + b * 128, 128)], bounce)
        pltpu.sync_copy(bounce, out_hbm.at[cid, pl.ds(sid * SEG + b * 128, 128)])


def _edge_pipeline(ids_s, ids_d, st_hbm, out_hbm, srows2, msg2, bounce, acc,
                   gs, sc, cid, sid, compute, prefetch):
    def gdesc(b, j):
        return pltpu.make_async_copy(st_hbm.at[ids_s.at[j]], srows2[b], gs[b])

    def sdesc(b, j):
        return pltpu.make_async_copy(msg2[b], acc.at[ids_d.at[j]], sc[b])

    for b in range(2):
        gdesc(b, b).start()
        prefetch(b, b, True)

    def step(jj, carry):
        for b in range(2):
            j = 2 * jj + b
            gdesc(b, j).wait()
            prefetch(b, j, False)

            @pl.when(jj > 0)
            def _():
                sdesc(b, j - 2).wait()

            compute(srows2[b], msg2[b], j)
            sdesc(b, j).start(add=True)

            @pl.when(j + 2 < NCH)
            def _():
                gdesc(b, j + 2).start()
                prefetch(b, j + 2, True)
        return carry

    lax.fori_loop(0, (NCH - 1) // 2, step, 0)
    gdesc(0, NCH - 1).wait()
    prefetch(0, NCH - 1, False)
    sdesc(0, NCH - 3).wait()
    compute(srows2[0], msg2[0], NCH - 1)
    sdesc(0, NCH - 1).start(add=True)
    sdesc(1, NCH - 2).wait()
    sdesc(0, NCH - 1).wait()

    plsc.subcore_barrier()
    _dump_acc(bounce, acc, out_hbm, cid, sid)


def _edge1_body(src3, dst3, st_hbm, dt_hbm, out_hbm,
                ids_s, ids_d, sr_a, sr_b, dr_a, dr_b, ms_a, ms_b,
                bounce, acc, gs_a, gs_b, gd_a, gd_b, sc_a, sc_b):
    cid = lax.axis_index("c")
    sid = lax.axis_index("s")
    wid = cid * NS + sid
    _zero_acc(bounce, acc, sid)
    pltpu.sync_copy(src3.at[wid], ids_s)
    pltpu.sync_copy(dst3.at[wid], ids_d)
    plsc.subcore_barrier()

    iota = lax.iota(jnp.int32, 16)
    drows2 = [dr_a, dr_b]
    gd = [gd_a, gd_b]

    def prefetch(b, j, start):
        d = pltpu.make_async_copy(dt_hbm.at[ids_d.at[j]], drows2[b], gd[b])
        if start:
            d.start()
        else:
            d.wait()

    def compute_fn(srows, msg, j):
        drows = dr_a if srows is sr_a else dr_b

        def grp(g, carry):
            rows = 16 * g + iota
            for h in range(H1):
                s_h = plsc.load_gather(srows, [rows, jnp.full((16,), 64 + h, jnp.int32)])
                d_h = plsc.load_gather(drows, [rows, jnp.full((16,), h, jnp.int32)])
                m_h = plsc.load_gather(drows, [rows, jnp.full((16,), 8 + h, jnp.int32)])
                z = s_h + d_h
                exh = jnp.exp(jnp.maximum(z, 0.2 * z) - m_h)
                plsc.store_scatter(msg, [rows, jnp.full((16,), 64 + h, jnp.int32)], exh)
                for f in range(F1):
                    cf = jnp.full((16,), F1 * h + f, jnp.int32)
                    col = plsc.load_gather(srows, [rows, cf])
                    plsc.store_scatter(msg, [rows, cf], exh * col)
            return carry

        lax.fori_loop(0, CH // 16, grp, 0)

    _edge_pipeline(ids_s, ids_d, st_hbm, out_hbm, [sr_a, sr_b], [ms_a, ms_b],
                   bounce, acc, [gs_a, gs_b], [sc_a, sc_b], cid, sid,
                   compute_fn, prefetch)


def _edge2_body(src3, dst3, st_hbm, v3_hbm, out_hbm,
                ids_s, ids_d, sr_a, sr_b, ms_a, ms_b,
                asrc_t, adst_t, mub_t,
                bounce, acc, gs_a, gs_b, sc_a, sc_b):
    cid = lax.axis_index("c")
    sid = lax.axis_index("s")
    wid = cid * NS + sid
    _zero_acc(bounce, acc, sid)
    pltpu.sync_copy(src3.at[wid], ids_s)
    pltpu.sync_copy(dst3.at[wid], ids_d)
    pltpu.sync_copy(v3_hbm.at[0], asrc_t)
    pltpu.sync_copy(v3_hbm.at[1], adst_t)
    pltpu.sync_copy(v3_hbm.at[2], mub_t)
    plsc.subcore_barrier()

    iota = lax.iota(jnp.int32, 16)
    zero16 = jnp.zeros((16,), jnp.float32)

    def zpad(i, carry):
        for m in (ms_a, ms_b):
            m[i, pl.ds(AR2 - 16, 16)] = zero16
        return carry

    lax.fori_loop(0, CH, zpad, 0)

    def prefetch(b, j, start):
        pass

    def compute_fn(srows, msg, j):
        def grp(g, carry):
            ivs = ids_s[j, pl.ds(16 * g, 16)]
            ivd = ids_d[j, pl.ds(16 * g, 16)]
            rows = 16 * g + iota
            s16 = plsc.load_gather(asrc_t, [ivs])
            d16 = plsc.load_gather(adst_t, [ivd])
            m16 = plsc.load_gather(mub_t, [ivd])
            z = s16 + d16
            ex = jnp.exp(jnp.maximum(z, 0.2 * z) - m16)
            plsc.store_scatter(msg, [rows, jnp.full((16,), C2, jnp.int32)], ex)
            for f in range(C2):
                cf = jnp.full((16,), f, jnp.int32)
                col = plsc.load_gather(srows, [rows, cf])
                plsc.store_scatter(msg, [rows, cf], ex * col)
            return carry

        lax.fori_loop(0, CH // 16, grp, 0)

    _edge_pipeline(ids_s, ids_d, st_hbm, out_hbm, [sr_a, sr_b], [ms_a, ms_b],
                   bounce, acc, [gs_a, gs_b], [sc_a, sc_b], cid, sid,
                   compute_fn, prefetch)


_edge1 = functools.partial(
    pl.kernel, _edge1_body,
    out_type=jax.ShapeDtypeStruct((NC, NP, SR1), jnp.float32),
    mesh=_mesh,
    compiler_params=_params,
    scratch_types=[
        pltpu.VMEM((NCH, CH), jnp.int32),
        pltpu.VMEM((NCH, CH), jnp.int32),
        pltpu.VMEM((CH, SR1), jnp.float32),
        pltpu.VMEM((CH, SR1), jnp.float32),
        pltpu.VMEM((CH, 16), jnp.float32),
        pltpu.VMEM((CH, 16), jnp.float32),
        pltpu.VMEM((CH, SR1), jnp.float32),
        pltpu.VMEM((CH, SR1), jnp.float32),
        pltpu.VMEM((128, SR1), jnp.float32),
        pltpu.VMEM_SHARED((NP, SR1), jnp.float32),
        pltpu.SemaphoreType.DMA,
        pltpu.SemaphoreType.DMA,
        pltpu.SemaphoreType.DMA,
        pltpu.SemaphoreType.DMA,
        pltpu.SemaphoreType.DMA,
        pltpu.SemaphoreType.DMA,
    ],
)()

_edge2 = functools.partial(
    pl.kernel, _edge2_body,
    out_type=jax.ShapeDtypeStruct((NC, NP, AR2), jnp.float32),
    mesh=_mesh,
    compiler_params=_params,
    scratch_types=[
        pltpu.VMEM((NCH, CH), jnp.int32),
        pltpu.VMEM((NCH, CH), jnp.int32),
        pltpu.VMEM((CH, SR2), jnp.float32),
        pltpu.VMEM((CH, SR2), jnp.float32),
        pltpu.VMEM((CH, AR2), jnp.float32),
        pltpu.VMEM((CH, AR2), jnp.float32),
        pltpu.VMEM((N,), jnp.float32),
        pltpu.VMEM((N,), jnp.float32),
        pltpu.VMEM((N,), jnp.float32),
        pltpu.VMEM((128, AR2), jnp.float32),
        pltpu.VMEM_SHARED((NP, AR2), jnp.float32),
        pltpu.SemaphoreType.DMA,
        pltpu.SemaphoreType.DMA,
        pltpu.SemaphoreType.DMA,
        pltpu.SemaphoreType.DMA,
    ],
)()


def kernel(x, edge_index, W1, a_src1, a_dst1, W2, a_src2, a_dst2):
    src3 = edge_index[0].reshape(NW, NCH, CH)
    dst3 = edge_index[1].reshape(NW, NCH, CH)

    ii = jnp.arange(H1 * F1)
    asb = jnp.zeros((H1 * F1, H1), jnp.float32).at[ii, ii // F1].set(a_src1.reshape(-1))
    adb = jnp.zeros((H1 * F1, H1), jnp.float32).at[ii, ii // F1].set(a_dst1.reshape(-1))
    rep = jnp.zeros((H1, H1 * F1), jnp.float32).at[ii // F1, ii].set(1.0)

    st1, dt1 = pl.pallas_call(
        _tabs1_body,
        out_shape=(
            jax.ShapeDtypeStruct((N, SR1), jnp.float32),
            jax.ShapeDtypeStruct((N, 16), jnp.float32),
        ),
    )(x, W1, asb, adb)

    acc1 = _edge1(src3, dst3, st1, dt1)

    st2, v3 = pl.pallas_call(
        _tabs2_body,
        out_shape=(
            jax.ShapeDtypeStruct((N, SR2), jnp.float32),
            jax.ShapeDtypeStruct((3, N), jnp.float32),
        ),
    )(acc1, W2, a_src2, a_dst2, rep)

    acc2 = _edge2(src3, dst3, st2, v3)

    out = pl.pallas_call(
        _final_body,
        out_shape=jax.ShapeDtypeStruct((N, C2), jnp.float32),
    )(acc2)
    return out


# speedup vs baseline: 77.6880x; 1.1796x over previous
import functools

import jax
import jax.numpy as jnp
from jax import lax
from jax.experimental import pallas as pl
from jax.experimental.pallas import tpu as pltpu
from jax.experimental.pallas import tpu_sc as plsc

N = 10000
E = 320000
D = 128
H1 = 8
F1 = 8
C2 = 16

NC = 2
NS = 16
NW = NC * NS
EW = E // NW
CH = 80
NCH = EW // CH
NP = 10240
SEG = NP // NS
SR1 = 80
SR2 = 16
AR2 = 32

_mesh = plsc.VectorSubcoreMesh(
    core_axis_name="c", subcore_axis_name="s", num_cores=NC, num_subcores=NS)
_params = pltpu.CompilerParams(
    needs_layout_passes=False, use_tc_tiling_on_sc=False)


def _tabs1_body(x_ref, w_ref, asb_ref, adb_ref, st_ref, dt_ref):
    h = jnp.dot(x_ref[...], w_ref[...], preferred_element_type=jnp.float32)
    asrc = jnp.dot(h, asb_ref[...], preferred_element_type=jnp.float32)
    adst = jnp.dot(h, adb_ref[...], preferred_element_type=jnp.float32)
    t = jnp.max(asrc, axis=0, keepdims=True) + adst
    mub = jnp.where(t > 0, t, 0.2 * t)
    st_ref[...] = jnp.concatenate([h, asrc], axis=1)
    dt_ref[...] = jnp.concatenate([adst, mub], axis=1)


def _tabs2_body(acc_ref, w2_ref, as2_ref, ad2_ref, rep_ref, st_ref, v3_ref):
    acc = acc_ref[0][:N] + acc_ref[1][:N]
    num = acc[:, 0:64]
    den = jnp.dot(acc[:, 64:72], rep_ref[...], preferred_element_type=jnp.float32)
    h1 = num / (den + 1e-9)
    h1 = jnp.where(h1 > 0, h1, jnp.exp(jnp.minimum(h1, 0.0)) - 1.0)
    h2 = jnp.dot(h1, w2_ref[...], preferred_element_type=jnp.float32)
    asrc2 = lax.dot_general(as2_ref[...], h2, (((1,), (1,)), ((), ())),
                            preferred_element_type=jnp.float32)
    adst2 = lax.dot_general(ad2_ref[...], h2, (((1,), (1,)), ((), ())),
                            preferred_element_type=jnp.float32)
    t = jnp.max(asrc2, axis=1, keepdims=True) + adst2
    mub2 = jnp.where(t > 0, t, 0.2 * t)
    st_ref[...] = h2
    v3_ref[...] = jnp.concatenate([asrc2, adst2, mub2], axis=0)


def _final_body(acc_ref, out_ref):
    acc = acc_ref[0][:N] + acc_ref[1][:N]
    out_ref[...] = acc[:, 0:C2] / (acc[:, C2:C2 + 1] + 1e-9)


def _zero_acc(bounce, acc, sid):
    zero16 = jnp.zeros((16,), jnp.float32)
    width = bounce.shape[1]

    def zrow(i, carry):
        for k in range(width // 16):
            bounce[i, pl.ds(16 * k, 16)] = zero16
        return carry

    lax.fori_loop(0, 128, zrow, 0)
    for b in range(5):
        pltpu.sync_copy(bounce, acc.at[pl.ds(sid * SEG + b * 128, 128)])


def _dump_acc(bounce, acc, out_hbm, cid, sid):
    for b in range(5):
        pltpu.sync_copy(acc.at[pl.ds(sid * SEG + b * 128, 128)], bounce)
        pltpu.sync_copy(bounce, out_hbm.at[cid, pl.ds(sid * SEG + b * 128, 128)])


def _edge_pipeline(ids_s, ids_d, st_hbm, out_hbm, srows2, msg2, bounce, acc,
                   gs, sc, cid, sid, compute, prefetch):
    def gdesc(b, j):
        return pltpu.make_async_copy(st_hbm.at[ids_s.at[j]], srows2[b], gs[b])

    def sdesc(b, j):
        return pltpu.make_async_copy(msg2[b], acc.at[ids_d.at[j]], sc[b])

    for b in range(2):
        gdesc(b, b).start()
        prefetch(b, b, True)

    def step(jj, carry):
        for b in range(2):
            j = 2 * jj + b
            gdesc(b, j).wait()
            prefetch(b, j, False)

            @pl.when(jj > 0)
            def _():
                sdesc(b, j - 2).wait()

            compute(srows2[b], msg2[b], j)
            sdesc(b, j).start(add=True)

            @pl.when(j + 2 < NCH)
            def _():
                gdesc(b, j + 2).start()
                prefetch(b, j + 2, True)
        return carry

    lax.fori_loop(0, (NCH - 1) // 2, step, 0)
    gdesc(0, NCH - 1).wait()
    prefetch(0, NCH - 1, False)
    sdesc(0, NCH - 3).wait()
    compute(srows2[0], msg2[0], NCH - 1)
    sdesc(0, NCH - 1).start(add=True)
    sdesc(1, NCH - 2).wait()
    sdesc(0, NCH - 1).wait()

    plsc.subcore_barrier()
    _dump_acc(bounce, acc, out_hbm, cid, sid)


def _edge1_body(src3, dst3, st_hbm, dt_hbm, out_hbm,
                ids_s, ids_d, sr_a, sr_b, dr_a, dr_b, ms_a, ms_b,
                bounce, acc, gs_a, gs_b, gd_a, gd_b, sc_a, sc_b):
    cid = lax.axis_index("c")
    sid = lax.axis_index("s")
    wid = cid * NS + sid
    _zero_acc(bounce, acc, sid)
    pltpu.sync_copy(src3.at[wid], ids_s)
    pltpu.sync_copy(dst3.at[wid], ids_d)
    plsc.subcore_barrier()

    iota = lax.iota(jnp.int32, 16)
    drows2 = [dr_a, dr_b]
    gd = [gd_a, gd_b]

    def prefetch(b, j, start):
        d = pltpu.make_async_copy(dt_hbm.at[ids_d.at[j]], drows2[b], gd[b])
        if start:
            d.start()
        else:
            d.wait()

    def compute_fn(srows, msg, j):
        drows = dr_a if srows is sr_a else dr_b

        def grp(g, carry):
            rows = 16 * g + iota
            for h in range(H1):
                s_h = plsc.bitcast(
                    plsc.load_gather(srows, [rows, jnp.full((16,), 32 + h, jnp.int32)]),
                    jnp.float32)
                d_h = plsc.load_gather(drows, [rows, jnp.full((16,), h, jnp.int32)])
                m_h = plsc.load_gather(drows, [rows, jnp.full((16,), 8 + h, jnp.int32)])
                z = s_h + d_h
                exh = jnp.exp(jnp.maximum(z, 0.2 * z) - m_h)
                plsc.store_scatter(msg, [rows, jnp.full((16,), 64 + h, jnp.int32)], exh)
                for c in range(4):
                    w = plsc.load_gather(srows, [rows, jnp.full((16,), 4 * h + c, jnp.int32)])
                    flo = plsc.bitcast(w << 16, jnp.float32)
                    fhi = plsc.bitcast(w & jnp.int32(-65536), jnp.float32)
                    plsc.store_scatter(
                        msg, [rows, jnp.full((16,), F1 * h + 2 * c, jnp.int32)], exh * flo)
                    plsc.store_scatter(
                        msg, [rows, jnp.full((16,), F1 * h + 2 * c + 1, jnp.int32)], exh * fhi)
            return carry

        lax.fori_loop(0, CH // 16, grp, 0)

    _edge_pipeline(ids_s, ids_d, st_hbm, out_hbm, [sr_a, sr_b], [ms_a, ms_b],
                   bounce, acc, [gs_a, gs_b], [sc_a, sc_b], cid, sid,
                   compute_fn, prefetch)


def _edge2_body(src3, dst3, st_hbm, v3_hbm, out_hbm,
                ids_s, ids_d, sr_a, sr_b, ms_a, ms_b,
                asrc_t, adst_t, mub_t,
                bounce, acc, gs_a, gs_b, sc_a, sc_b):
    cid = lax.axis_index("c")
    sid = lax.axis_index("s")
    wid = cid * NS + sid
    _zero_acc(bounce, acc, sid)
    pltpu.sync_copy(src3.at[wid], ids_s)
    pltpu.sync_copy(dst3.at[wid], ids_d)
    pltpu.sync_copy(v3_hbm.at[0], asrc_t)
    pltpu.sync_copy(v3_hbm.at[1], adst_t)
    pltpu.sync_copy(v3_hbm.at[2], mub_t)
    plsc.subcore_barrier()

    iota = lax.iota(jnp.int32, 16)
    zero16 = jnp.zeros((16,), jnp.float32)

    def zpad(i, carry):
        for m in (ms_a, ms_b):
            m[i, pl.ds(AR2 - 16, 16)] = zero16
        return carry

    lax.fori_loop(0, CH, zpad, 0)

    def prefetch(b, j, start):
        pass

    def compute_fn(srows, msg, j):
        def grp(g, carry):
            ivs = ids_s[j, pl.ds(16 * g, 16)]
            ivd = ids_d[j, pl.ds(16 * g, 16)]
            rows = 16 * g + iota
            s16 = plsc.load_gather(asrc_t, [ivs])
            d16 = plsc.load_gather(adst_t, [ivd])
            m16 = plsc.load_gather(mub_t, [ivd])
            z = s16 + d16
            ex = jnp.exp(jnp.maximum(z, 0.2 * z) - m16)
            plsc.store_scatter(msg, [rows, jnp.full((16,), C2, jnp.int32)], ex)
            for f in range(C2):
                cf = jnp.full((16,), f, jnp.int32)
                col = plsc.load_gather(srows, [rows, cf])
                plsc.store_scatter(msg, [rows, cf], ex * col)
            return carry

        lax.fori_loop(0, CH // 16, grp, 0)

    _edge_pipeline(ids_s, ids_d, st_hbm, out_hbm, [sr_a, sr_b], [ms_a, ms_b],
                   bounce, acc, [gs_a, gs_b], [sc_a, sc_b], cid, sid,
                   compute_fn, prefetch)


_edge1 = functools.partial(
    pl.kernel, _edge1_body,
    out_type=jax.ShapeDtypeStruct((NC, NP, SR1), jnp.float32),
    mesh=_mesh,
    compiler_params=_params,
    scratch_types=[
        pltpu.VMEM((NCH, CH), jnp.int32),
        pltpu.VMEM((NCH, CH), jnp.int32),
        pltpu.VMEM((CH, 48), jnp.int32),
        pltpu.VMEM((CH, 48), jnp.int32),
        pltpu.VMEM((CH, 16), jnp.float32),
        pltpu.VMEM((CH, 16), jnp.float32),
        pltpu.VMEM((CH, SR1), jnp.float32),
        pltpu.VMEM((CH, SR1), jnp.float32),
        pltpu.VMEM((128, SR1), jnp.float32),
        pltpu.VMEM_SHARED((NP, SR1), jnp.float32),
        pltpu.SemaphoreType.DMA,
        pltpu.SemaphoreType.DMA,
        pltpu.SemaphoreType.DMA,
        pltpu.SemaphoreType.DMA,
        pltpu.SemaphoreType.DMA,
        pltpu.SemaphoreType.DMA,
    ],
)()

_edge2 = functools.partial(
    pl.kernel, _edge2_body,
    out_type=jax.ShapeDtypeStruct((NC, NP, AR2), jnp.float32),
    mesh=_mesh,
    compiler_params=_params,
    scratch_types=[
        pltpu.VMEM((NCH, CH), jnp.int32),
        pltpu.VMEM((NCH, CH), jnp.int32),
        pltpu.VMEM((CH, SR2), jnp.float32),
        pltpu.VMEM((CH, SR2), jnp.float32),
        pltpu.VMEM((CH, AR2), jnp.float32),
        pltpu.VMEM((CH, AR2), jnp.float32),
        pltpu.VMEM((N,), jnp.float32),
        pltpu.VMEM((N,), jnp.float32),
        pltpu.VMEM((N,), jnp.float32),
        pltpu.VMEM((128, AR2), jnp.float32),
        pltpu.VMEM_SHARED((NP, AR2), jnp.float32),
        pltpu.SemaphoreType.DMA,
        pltpu.SemaphoreType.DMA,
        pltpu.SemaphoreType.DMA,
        pltpu.SemaphoreType.DMA,
    ],
)()


def kernel(x, edge_index, W1, a_src1, a_dst1, W2, a_src2, a_dst2):
    src3 = edge_index[0].reshape(NW, NCH, CH)
    dst3 = edge_index[1].reshape(NW, NCH, CH)

    ii = jnp.arange(H1 * F1)
    asb = jnp.zeros((H1 * F1, H1), jnp.float32).at[ii, ii // F1].set(a_src1.reshape(-1))
    adb = jnp.zeros((H1 * F1, H1), jnp.float32).at[ii, ii // F1].set(a_dst1.reshape(-1))
    rep = jnp.zeros((H1, H1 * F1), jnp.float32).at[ii // F1, ii].set(1.0)

    stf, dt1 = pl.pallas_call(
        _tabs1_body,
        out_shape=(
            jax.ShapeDtypeStruct((N, 72), jnp.float32),
            jax.ShapeDtypeStruct((N, 16), jnp.float32),
        ),
    )(x, W1, asb, adb)
    hp = lax.bitcast_convert_type(
        stf[:, :64].astype(jnp.bfloat16).reshape(N, 32, 2), jnp.int32)
    ab = lax.bitcast_convert_type(stf[:, 64:72], jnp.int32)
    st1 = jnp.concatenate([hp, ab, jnp.zeros((N, 8), jnp.int32)], axis=1)

    acc1 = _edge1(src3, dst3, st1, dt1)

    st2, v3 = pl.pallas_call(
        _tabs2_body,
        out_shape=(
            jax.ShapeDtypeStruct((N, SR2), jnp.float32),
            jax.ShapeDtypeStruct((3, N), jnp.float32),
        ),
    )(acc1, W2, a_src2, a_dst2, rep)

    acc2 = _edge2(src3, dst3, st2, v3)

    out = pl.pallas_call(
        _final_body,
        out_shape=jax.ShapeDtypeStruct((N, C2), jnp.float32),
    )(acc2)
    return out


# speedup vs baseline: 85.2432x; 1.0973x over previous
import functools

import jax
import jax.numpy as jnp
from jax import lax
from jax.experimental import pallas as pl
from jax.experimental.pallas import tpu as pltpu
from jax.experimental.pallas import tpu_sc as plsc

N = 10000
E = 320000
D = 128
H1 = 8
F1 = 8
C2 = 16

NC = 2
NS = 16
NW = NC * NS
EW = E // NW
CH = 80
NCH = EW // CH
NP = 10240
SEG = NP // NS
SR1 = 80
SR2 = 16
AR2 = 16

_mesh = plsc.VectorSubcoreMesh(
    core_axis_name="c", subcore_axis_name="s", num_cores=NC, num_subcores=NS)
_params = pltpu.CompilerParams(
    needs_layout_passes=False, use_tc_tiling_on_sc=False)


def _tabs1_body(x_ref, w_ref, asb_ref, adb_ref, st_ref, dt_ref):
    h = jnp.dot(x_ref[...], w_ref[...], preferred_element_type=jnp.float32)
    asrc = jnp.dot(h, asb_ref[...], preferred_element_type=jnp.float32)
    adst = jnp.dot(h, adb_ref[...], preferred_element_type=jnp.float32)
    t = jnp.max(asrc, axis=0, keepdims=True) + adst
    mub = jnp.where(t > 0, t, 0.2 * t)
    st_ref[...] = jnp.concatenate([h, asrc], axis=1)
    dt_ref[...] = jnp.concatenate([adst, mub], axis=1)


def _tabs2_body(acc_ref, w2_ref, as2_ref, ad2_ref, rep_ref, st_ref, v3_ref):
    acc = acc_ref[0][:N] + acc_ref[1][:N]
    num = acc[:, 0:64]
    den = jnp.dot(acc[:, 64:72], rep_ref[...], preferred_element_type=jnp.float32)
    h1 = num / (den + 1e-9)
    h1 = jnp.where(h1 > 0, h1, jnp.exp(jnp.minimum(h1, 0.0)) - 1.0)
    h2 = jnp.dot(h1, w2_ref[...], preferred_element_type=jnp.float32)
    asrc2 = lax.dot_general(as2_ref[...], h2, (((1,), (1,)), ((), ())),
                            preferred_element_type=jnp.float32)
    adst2 = lax.dot_general(ad2_ref[...], h2, (((1,), (1,)), ((), ())),
                            preferred_element_type=jnp.float32)
    t = jnp.max(asrc2, axis=1, keepdims=True) + adst2
    mub2 = jnp.where(t > 0, t, 0.2 * t)
    st_ref[...] = h2
    v3_ref[...] = jnp.concatenate([asrc2, adst2, mub2], axis=0)


def _final_body(acc_ref, den_ref, ones_ref, out_ref):
    acc = acc_ref[0][:N] + acc_ref[1][:N]
    den = lax.dot_general(den_ref[...], ones_ref[...], (((0,), (0,)), ((), ())),
                          preferred_element_type=jnp.float32)
    out_ref[...] = acc[:, 0:C2] / (den + 1e-9)


def _zero_acc(bounce, acc, sid):
    zero16 = jnp.zeros((16,), jnp.float32)
    width = bounce.shape[1]

    def zrow(i, carry):
        for k in range(width // 16):
            bounce[i, pl.ds(16 * k, 16)] = zero16
        return carry

    lax.fori_loop(0, 128, zrow, 0)
    for b in range(5):
        pltpu.sync_copy(bounce, acc.at[pl.ds(sid * SEG + b * 128, 128)])


def _dump_acc(bounce, acc, out_hbm, cid, sid):
    for b in range(5):
        pltpu.sync_copy(acc.at[pl.ds(sid * SEG + b * 128, 128)], bounce)
        pltpu.sync_copy(bounce, out_hbm.at[cid, pl.ds(sid * SEG + b * 128, 128)])


def _edge_pipeline(ids_s, ids_d, st_hbm, out_hbm, srows2, msg2, bounce, acc,
                   gs, sc, cid, sid, compute, prefetch):
    def gdesc(b, j):
        return pltpu.make_async_copy(st_hbm.at[ids_s.at[j]], srows2[b], gs[b])

    def sdesc(b, j):
        return pltpu.make_async_copy(msg2[b], acc.at[ids_d.at[j]], sc[b])

    for b in range(2):
        gdesc(b, b).start()
        prefetch(b, b, True)

    def step(jj, carry):
        for b in range(2):
            j = 2 * jj + b
            gdesc(b, j).wait()
            prefetch(b, j, False)

            @pl.when(jj > 0)
            def _():
                sdesc(b, j - 2).wait()

            compute(srows2[b], msg2[b], j)
            sdesc(b, j).start(add=True)

            @pl.when(j + 2 < NCH)
            def _():
                gdesc(b, j + 2).start()
                prefetch(b, j + 2, True)
        return carry

    lax.fori_loop(0, (NCH - 1) // 2, step, 0)
    gdesc(0, NCH - 1).wait()
    prefetch(0, NCH - 1, False)
    sdesc(0, NCH - 3).wait()
    compute(srows2[0], msg2[0], NCH - 1)
    sdesc(0, NCH - 1).start(add=True)
    sdesc(1, NCH - 2).wait()
    sdesc(0, NCH - 1).wait()

    plsc.subcore_barrier()
    _dump_acc(bounce, acc, out_hbm, cid, sid)


def _edge1_body(src3, dst3, st_hbm, dt_hbm, out_hbm,
                ids_s, ids_d, sr_a, sr_b, dr_a, dr_b, ms_a, ms_b,
                bounce, acc, gs_a, gs_b, gd_a, gd_b, sc_a, sc_b):
    cid = lax.axis_index("c")
    sid = lax.axis_index("s")
    wid = cid * NS + sid
    _zero_acc(bounce, acc, sid)
    pltpu.sync_copy(src3.at[wid], ids_s)
    pltpu.sync_copy(dst3.at[wid], ids_d)
    plsc.subcore_barrier()

    iota = lax.iota(jnp.int32, 16)
    drows2 = [dr_a, dr_b]
    gd = [gd_a, gd_b]

    def prefetch(b, j, start):
        d = pltpu.make_async_copy(dt_hbm.at[ids_d.at[j]], drows2[b], gd[b])
        if start:
            d.start()
        else:
            d.wait()

    def compute_fn(srows, msg, j):
        drows = dr_a if srows is sr_a else dr_b

        def grp(g, carry):
            rows = 16 * g + iota
            for h in range(H1):
                s_h = plsc.bitcast(
                    plsc.load_gather(srows, [rows, jnp.full((16,), 32 + h, jnp.int32)]),
                    jnp.float32)
                d_h = plsc.load_gather(drows, [rows, jnp.full((16,), h, jnp.int32)])
                m_h = plsc.load_gather(drows, [rows, jnp.full((16,), 8 + h, jnp.int32)])
                z = s_h + d_h
                exh = jnp.exp(jnp.maximum(z, 0.2 * z) - m_h)
                plsc.store_scatter(msg, [rows, jnp.full((16,), 64 + h, jnp.int32)], exh)
                for c in range(4):
                    w = plsc.load_gather(srows, [rows, jnp.full((16,), 4 * h + c, jnp.int32)])
                    flo = plsc.bitcast(w << 16, jnp.float32)
                    fhi = plsc.bitcast(w & jnp.int32(-65536), jnp.float32)
                    plsc.store_scatter(
                        msg, [rows, jnp.full((16,), F1 * h + 2 * c, jnp.int32)], exh * flo)
                    plsc.store_scatter(
                        msg, [rows, jnp.full((16,), F1 * h + 2 * c + 1, jnp.int32)], exh * fhi)
            return carry

        lax.fori_loop(0, CH // 16, grp, 0)

    _edge_pipeline(ids_s, ids_d, st_hbm, out_hbm, [sr_a, sr_b], [ms_a, ms_b],
                   bounce, acc, [gs_a, gs_b], [sc_a, sc_b], cid, sid,
                   compute_fn, prefetch)


def _edge2_body(src3, dst3, st_hbm, v3_hbm, out_hbm, outd_hbm,
                ids_s, ids_d, sr_a, sr_b, ms_a, ms_b,
                asrc_t, adst_t, mub_t, den_t,
                bounce, acc, gs_a, gs_b, sc_a, sc_b):
    cid = lax.axis_index("c")
    sid = lax.axis_index("s")
    wid = cid * NS + sid
    _zero_acc(bounce, acc, sid)
    pltpu.sync_copy(src3.at[wid], ids_s)
    pltpu.sync_copy(dst3.at[wid], ids_d)
    pltpu.sync_copy(v3_hbm.at[0], asrc_t)
    pltpu.sync_copy(v3_hbm.at[1], adst_t)
    pltpu.sync_copy(v3_hbm.at[2], mub_t)

    def zden(i, carry):
        den_t[pl.ds(16 * i, 16)] = jnp.zeros((16,), jnp.float32)
        return carry

    lax.fori_loop(0, N // 16, zden, 0)
    plsc.subcore_barrier()

    iota = lax.iota(jnp.int32, 16)

    def prefetch(b, j, start):
        pass

    def compute_fn(srows, msg, j):
        def grp(g, carry):
            ivs = ids_s[j, pl.ds(16 * g, 16)]
            ivd = ids_d[j, pl.ds(16 * g, 16)]
            rows = 16 * g + iota
            s16 = plsc.load_gather(asrc_t, [ivs])
            d16 = plsc.load_gather(adst_t, [ivd])
            m16 = plsc.load_gather(mub_t, [ivd])
            z = s16 + d16
            ex = jnp.exp(jnp.maximum(z, 0.2 * z) - m16)
            plsc.addupdate_scatter(den_t, [ivd], ex)
            for f in range(C2):
                cf = jnp.full((16,), f, jnp.int32)
                col = plsc.load_gather(srows, [rows, cf])
                plsc.store_scatter(msg, [rows, cf], ex * col)
            return carry

        lax.fori_loop(0, CH // 16, grp, 0)

    _edge_pipeline(ids_s, ids_d, st_hbm, out_hbm, [sr_a, sr_b], [ms_a, ms_b],
                   bounce, acc, [gs_a, gs_b], [sc_a, sc_b], cid, sid,
                   compute_fn, prefetch)
    pltpu.sync_copy(den_t, outd_hbm.at[wid])


_edge1 = functools.partial(
    pl.kernel, _edge1_body,
    out_type=jax.ShapeDtypeStruct((NC, NP, SR1), jnp.float32),
    mesh=_mesh,
    compiler_params=_params,
    scratch_types=[
        pltpu.VMEM((NCH, CH), jnp.int32),
        pltpu.VMEM((NCH, CH), jnp.int32),
        pltpu.VMEM((CH, 48), jnp.int32),
        pltpu.VMEM((CH, 48), jnp.int32),
        pltpu.VMEM((CH, 16), jnp.float32),
        pltpu.VMEM((CH, 16), jnp.float32),
        pltpu.VMEM((CH, SR1), jnp.float32),
        pltpu.VMEM((CH, SR1), jnp.float32),
        pltpu.VMEM((128, SR1), jnp.float32),
        pltpu.VMEM_SHARED((NP, SR1), jnp.float32),
        pltpu.SemaphoreType.DMA,
        pltpu.SemaphoreType.DMA,
        pltpu.SemaphoreType.DMA,
        pltpu.SemaphoreType.DMA,
        pltpu.SemaphoreType.DMA,
        pltpu.SemaphoreType.DMA,
    ],
)()

_edge2 = functools.partial(
    pl.kernel, _edge2_body,
    out_type=(jax.ShapeDtypeStruct((NC, NP, AR2), jnp.float32),
              jax.ShapeDtypeStruct((NW, N), jnp.float32)),
    mesh=_mesh,
    compiler_params=_params,
    scratch_types=[
        pltpu.VMEM((NCH, CH), jnp.int32),
        pltpu.VMEM((NCH, CH), jnp.int32),
        pltpu.VMEM((CH, SR2), jnp.float32),
        pltpu.VMEM((CH, SR2), jnp.float32),
        pltpu.VMEM((CH, AR2), jnp.float32),
        pltpu.VMEM((CH, AR2), jnp.float32),
        pltpu.VMEM((N,), jnp.float32),
        pltpu.VMEM((N,), jnp.float32),
        pltpu.VMEM((N,), jnp.float32),
        pltpu.VMEM((N,), jnp.float32),
        pltpu.VMEM((128, AR2), jnp.float32),
        pltpu.VMEM_SHARED((NP, AR2), jnp.float32),
        pltpu.SemaphoreType.DMA,
        pltpu.SemaphoreType.DMA,
        pltpu.SemaphoreType.DMA,
        pltpu.SemaphoreType.DMA,
    ],
)()


def kernel(x, edge_index, W1, a_src1, a_dst1, W2, a_src2, a_dst2):
    src3 = edge_index[0].reshape(NW, NCH, CH)
    dst3 = edge_index[1].reshape(NW, NCH, CH)

    ii = jnp.arange(H1 * F1)
    asb = jnp.zeros((H1 * F1, H1), jnp.float32).at[ii, ii // F1].set(a_src1.reshape(-1))
    adb = jnp.zeros((H1 * F1, H1), jnp.float32).at[ii, ii // F1].set(a_dst1.reshape(-1))
    rep = jnp.zeros((H1, H1 * F1), jnp.float32).at[ii // F1, ii].set(1.0)

    stf, dt1 = pl.pallas_call(
        _tabs1_body,
        out_shape=(
            jax.ShapeDtypeStruct((N, 72), jnp.float32),
            jax.ShapeDtypeStruct((N, 16), jnp.float32),
        ),
    )(x, W1, asb, adb)
    hp = lax.bitcast_convert_type(
        stf[:, :64].astype(jnp.bfloat16).reshape(N, 32, 2), jnp.int32)
    ab = lax.bitcast_convert_type(stf[:, 64:72], jnp.int32)
    st1 = jnp.concatenate([hp, ab, jnp.zeros((N, 8), jnp.int32)], axis=1)

    acc1 = _edge1(src3, dst3, st1, dt1)

    st2, v3 = pl.pallas_call(
        _tabs2_body,
        out_shape=(
            jax.ShapeDtypeStruct((N, SR2), jnp.float32),
            jax.ShapeDtypeStruct((3, N), jnp.float32),
        ),
    )(acc1, W2, a_src2, a_dst2, rep)

    acc2, den2 = _edge2(src3, dst3, st2, v3)

    out = pl.pallas_call(
        _final_body,
        out_shape=jax.ShapeDtypeStruct((N, C2), jnp.float32),
    )(acc2, den2, jnp.ones((NW, 1), jnp.float32))
    return out
